# baseline (speedup 1.0000x reference)
import functools

import jax
import jax.numpy as jnp
from jax import lax
from jax.experimental import pallas as pl
from jax.experimental.pallas import tpu as pltpu
from jax.experimental.pallas import tpu_sc as plsc

_VOCAB = 1000000
_EMBD = 64
_BATCH = 16384
_NC = 2
_NS = 16
_NW = _NC * _NS
_BPW = _BATCH // _NW
_CHUNK = 128
_NCHUNK = _BPW // _CHUNK
_LANES = 16
_CONVW = 4096
_NGRP = _CONVW // _CHUNK
_NBLK = (_VOCAB + _CONVW - 1) // _CONVW
_PACKROWS = _NBLK * _CONVW // 2


def _conv_body(i_ref, o_ref):
    x = i_ref[...]
    col = lax.broadcasted_iota(jnp.int32, (_EMBD, _CHUNK), 1)
    row = lax.broadcasted_iota(jnp.int32, (_EMBD, _CHUNK), 0)
    ident0 = (col == row).astype(jnp.float32)
    ident1 = (col == row + _EMBD).astype(jnp.float32)
    pieces = []
    for q in range(_NGRP):
        base = q * _CHUNK
        lo = lax.dot_general(
            x[:, base:base + _EMBD], ident0,
            (((0,), (0,)), ((), ())), precision=lax.Precision.HIGHEST)
        hi = lax.dot_general(
            x[:, base + _EMBD:base + _CHUNK], ident1,
            (((0,), (0,)), ((), ())), precision=lax.Precision.HIGHEST)
        pieces.append(lo + hi)
    o_ref[...] = jnp.concatenate(pieces, axis=0)


_conv = pl.pallas_call(
    _conv_body,
    grid=(_NBLK,),
    in_specs=[pl.BlockSpec((_EMBD, _CONVW), lambda p: (0, p))],
    out_specs=pl.BlockSpec((_CONVW // 2, _CHUNK), lambda p: (p, 0)),
    out_shape=jax.ShapeDtypeStruct((_PACKROWS, _CHUNK), jnp.float32),
)


def _sc_body(fo_hbm, co_hbm, tab_hbm, out_hbm,
             fidx, cidx, frows, crows, accv, fsem, csem):
    wid = lax.axis_index("s") * _NC + lax.axis_index("c")
    pltpu.sync_copy(fo_hbm.at[wid], fidx)
    pltpu.sync_copy(co_hbm.at[wid], cidx)
    for k in range(_NCHUNK):
        for c in range(_CHUNK // _LANES):
            sl = pl.ds(c * _LANES, _LANES)
            i = fidx[k, sl]
            fidx[k, sl] = ((i >> 7) << 7) + ((i & 63) << 1) + ((i >> 6) & 1)
            i2 = cidx[k, sl]
            cidx[k, sl] = ((i2 >> 7) << 7) + ((i2 & 63) << 1) + ((i2 >> 6) & 1)
    acc = jnp.zeros((_LANES,), jnp.float32)
    for j in range(_NCHUNK):
        fcp = pltpu.async_copy(tab_hbm.at[fidx.at[j]], frows, fsem)
        ccp = pltpu.async_copy(tab_hbm.at[cidx.at[j]], crows, csem)
        fcp.wait()
        ccp.wait()

        def row(i, a):
            for c in range(_EMBD // _LANES):
                a = a + (frows[i, pl.ds(c * _LANES, _LANES)]
                         * crows[i, pl.ds(c * _LANES, _LANES)])
            return a

        acc = lax.fori_loop(0, _CHUNK, row, acc)
    accv[...] = acc
    pltpu.sync_copy(accv, out_hbm.at[pl.ds(wid * _LANES, _LANES)])


_sc_partials = functools.partial(
    pl.kernel,
    out_type=jax.ShapeDtypeStruct((_NW * _LANES,), jnp.float32),
    mesh=plsc.VectorSubcoreMesh(core_axis_name="c", subcore_axis_name="s"),
    scratch_types=[
        pltpu.VMEM((_NCHUNK, _CHUNK), jnp.int32),
        pltpu.VMEM((_NCHUNK, _CHUNK), jnp.int32),
        pltpu.VMEM((_CHUNK, _EMBD), jnp.float32),
        pltpu.VMEM((_CHUNK, _EMBD), jnp.float32),
        pltpu.VMEM((_LANES,), jnp.float32),
        pltpu.SemaphoreType.DMA,
        pltpu.SemaphoreType.DMA,
    ],
    compiler_params=pltpu.CompilerParams(use_tc_tiling_on_sc=False),
)(_sc_body)


def _finish_body(p_ref, o_ref):
    o_ref[...] = jax.nn.log_sigmoid(jnp.sum(p_ref[...])).reshape(1, 1)


_finish = pl.pallas_call(
    _finish_body,
    out_shape=jax.ShapeDtypeStruct((1, 1), jnp.float32),
)


def kernel(focus, context, embeddings):
    fo = focus.reshape(_NW, _NCHUNK, _CHUNK)
    co = context.reshape(_NW, _NCHUNK, _CHUNK)
    packed = _conv(embeddings.T)
    tab = packed.reshape(_PACKROWS * 2, _EMBD)
    partials = _sc_partials(fo, co, tab)
    return _finish(partials.reshape(4, 128))


# speedup vs baseline: 1.7651x; 1.7651x over previous
import functools

import jax
import jax.numpy as jnp
from jax import lax
from jax.experimental import pallas as pl
from jax.experimental.pallas import tpu as pltpu
from jax.experimental.pallas import tpu_sc as plsc

_VOCAB = 1000000
_EMBD = 64
_BATCH = 16384
_NC = 2
_NS = 16
_NW = _NC * _NS
_BPW = _BATCH // _NW
_CHUNK = 128
_NCHUNK = _BPW // _CHUNK
_LANES = 16
_CONVW = 4096
_NGRP = _CONVW // _CHUNK
_NBLK = (_VOCAB + _CONVW - 1) // _CONVW
_PACKROWS = _NBLK * _CONVW // 2


def _conv_body(i_ref, o_ref):
    xt = i_ref[...].T
    for q in range(_NGRP):
        base = q * _CHUNK
        o_ref[pl.ds(q * _EMBD, _EMBD), :] = jnp.concatenate(
            [xt[base:base + _EMBD, :], xt[base + _EMBD:base + _CHUNK, :]],
            axis=1)


_conv = pl.pallas_call(
    _conv_body,
    grid=(_NBLK,),
    in_specs=[pl.BlockSpec((_EMBD, _CONVW), lambda p: (0, p))],
    out_specs=pl.BlockSpec((_CONVW // 2, _CHUNK), lambda p: (p, 0)),
    out_shape=jax.ShapeDtypeStruct((_PACKROWS, _CHUNK), jnp.float32),
)


def _sc_body(fo_hbm, co_hbm, tab_hbm, out_hbm,
             fidx, cidx, frows, crows, accv, fsem, csem):
    wid = lax.axis_index("s") * _NC + lax.axis_index("c")
    pltpu.sync_copy(fo_hbm.at[wid], fidx)
    pltpu.sync_copy(co_hbm.at[wid], cidx)
    for k in range(_NCHUNK):
        for c in range(_CHUNK // _LANES):
            sl = pl.ds(c * _LANES, _LANES)
            i = fidx[k, sl]
            fidx[k, sl] = ((i >> 7) << 7) + ((i & 63) << 1) + ((i >> 6) & 1)
            i2 = cidx[k, sl]
            cidx[k, sl] = ((i2 >> 7) << 7) + ((i2 & 63) << 1) + ((i2 >> 6) & 1)
    acc = jnp.zeros((_LANES,), jnp.float32)
    for j in range(_NCHUNK):
        fcp = pltpu.async_copy(tab_hbm.at[fidx.at[j]], frows, fsem)
        ccp = pltpu.async_copy(tab_hbm.at[cidx.at[j]], crows, csem)
        fcp.wait()
        ccp.wait()

        def row(i, a):
            for c in range(_EMBD // _LANES):
                a = a + (frows[i, pl.ds(c * _LANES, _LANES)]
                         * crows[i, pl.ds(c * _LANES, _LANES)])
            return a

        acc = lax.fori_loop(0, _CHUNK, row, acc)
    accv[...] = acc
    pltpu.sync_copy(accv, out_hbm.at[pl.ds(wid * _LANES, _LANES)])


_sc_partials = functools.partial(
    pl.kernel,
    out_type=jax.ShapeDtypeStruct((_NW * _LANES,), jnp.float32),
    mesh=plsc.VectorSubcoreMesh(core_axis_name="c", subcore_axis_name="s"),
    scratch_types=[
        pltpu.VMEM((_NCHUNK, _CHUNK), jnp.int32),
        pltpu.VMEM((_NCHUNK, _CHUNK), jnp.int32),
        pltpu.VMEM((_CHUNK, _EMBD), jnp.float32),
        pltpu.VMEM((_CHUNK, _EMBD), jnp.float32),
        pltpu.VMEM((_LANES,), jnp.float32),
        pltpu.SemaphoreType.DMA,
        pltpu.SemaphoreType.DMA,
    ],
    compiler_params=pltpu.CompilerParams(use_tc_tiling_on_sc=False),
)(_sc_body)


def _finish_body(p_ref, o_ref):
    o_ref[...] = jax.nn.log_sigmoid(jnp.sum(p_ref[...])).reshape(1, 1)


_finish = pl.pallas_call(
    _finish_body,
    out_shape=jax.ShapeDtypeStruct((1, 1), jnp.float32),
)


def kernel(focus, context, embeddings):
    fo = focus.reshape(_NW, _NCHUNK, _CHUNK)
    co = context.reshape(_NW, _NCHUNK, _CHUNK)
    packed = _conv(embeddings.T)
    tab = packed.reshape(_PACKROWS * 2, _EMBD)
    partials = _sc_partials(fo, co, tab)
    return _finish(partials.reshape(4, 128))
